# baseline (speedup 1.0000x reference)
import functools

import jax
import jax.numpy as jnp
from jax.experimental import pallas as pl
from jax.experimental.pallas import tpu as pltpu


def _normalize(x):
    norm = jnp.sqrt(jnp.sum(x * x, axis=-1, keepdims=True))
    return x / jnp.maximum(norm, 1e-12)


def _topk_index_kernel(x_ref, idx_ref):
    x = x_ref[0]
    xn = _normalize(x)
    m = x.shape[0]
    adj = jax.lax.dot_general(xn, xn, (((1,), (1,)), ((), ())),
                              preferred_element_type=jnp.float32)
    row = jax.lax.broadcasted_iota(jnp.int32, adj.shape, 0)
    col = jax.lax.broadcasted_iota(jnp.int32, adj.shape, 1)
    adj = jnp.where(row == col, 0.0, adj)
    m1 = jnp.max(adj, axis=0, keepdims=True)
    i1 = jnp.min(jnp.where(adj == m1, row, m), axis=0, keepdims=True)
    masked = jnp.where(row == i1, -3.0, adj)
    m2 = jnp.max(masked, axis=0, keepdims=True)
    i2 = jnp.min(jnp.where(masked == m2, row, m), axis=0, keepdims=True)
    cols = jax.lax.broadcasted_iota(jnp.int32, i1.shape, 1)
    idx_ref[0, 0:1, :] = jnp.where(i1 == cols, m, i1)
    idx_ref[0, 1:2, :] = jnp.where(i2 == cols, m, i2)


def _output_kernel(rows, x_ref, idx_ref, o_ref):
    i = pl.program_id(1)
    x = x_ref[0]
    xn = _normalize(x)
    xr = _normalize(x_ref[0, pl.ds(i * rows, rows), :])
    adj = jax.lax.dot_general(xr, xn, (((1,), (1,)), ((), ())),
                              preferred_element_type=jnp.float32)
    row = jax.lax.broadcasted_iota(jnp.int32, adj.shape, 0)
    col = jax.lax.broadcasted_iota(jnp.int32, adj.shape, 1)
    i1c = idx_ref[0, 0:1, :] - i * rows
    i2c = idx_ref[0, 1:2, :] - i * rows
    i1r = jnp.swapaxes(idx_ref[0, 0:1, pl.ds(i * rows, rows)], 0, 1)
    i2r = jnp.swapaxes(idx_ref[0, 1:2, pl.ds(i * rows, rows)], 0, 1)
    ind_row = (col == i1r) | (col == i2r)
    ind_col = (row == i1c) | (row == i2c)
    h = adj * 0.5
    o_ref[0] = jnp.where(ind_row, h, 0.0) + jnp.where(ind_col, h, 0.0)


def kernel(x, W1, b1, W2, b2):
    b, m, d = x.shape
    rows = 512

    idx = pl.pallas_call(
        _topk_index_kernel,
        grid=(b,),
        in_specs=[pl.BlockSpec((1, m, d), lambda i: (i, 0, 0))],
        out_specs=pl.BlockSpec((1, 2, m), lambda i: (i, 0, 0)),
        out_shape=jax.ShapeDtypeStruct((b, 2, m), jnp.int32),
        compiler_params=pltpu.CompilerParams(
            dimension_semantics=("parallel",)),
    )(x)

    out = pl.pallas_call(
        functools.partial(_output_kernel, rows),
        grid=(b, m // rows),
        in_specs=[
            pl.BlockSpec((1, m, d), lambda i, j: (i, 0, 0)),
            pl.BlockSpec((1, 2, m), lambda i, j: (i, 0, 0)),
        ],
        out_specs=pl.BlockSpec((1, rows, m), lambda i, j: (i, j, 0)),
        out_shape=jax.ShapeDtypeStruct((b, m, m), jnp.float32),
        compiler_params=pltpu.CompilerParams(
            dimension_semantics=("parallel", "parallel")),
    )(x, idx)
    return out


# speedup vs baseline: 1.4998x; 1.4998x over previous
import functools

import jax
import jax.numpy as jnp
from jax.experimental import pallas as pl
from jax.experimental.pallas import tpu as pltpu


def _normalize(x):
    norm = jnp.sqrt(jnp.sum(x * x, axis=-1, keepdims=True))
    return x / jnp.maximum(norm, 1e-12)


def _fused_kernel(rows, x_ref, o_ref, adj_s, tc_s, tr_s):
    j = pl.program_id(1)

    @pl.when(j == 0)
    def _prologue():
        xn = _normalize(x_ref[0])
        adj = jax.lax.dot_general(xn, xn, (((1,), (1,)), ((), ())),
                                  preferred_element_type=jnp.float32)
        row = jax.lax.broadcasted_iota(jnp.int32, adj.shape, 0)
        col = jax.lax.broadcasted_iota(jnp.int32, adj.shape, 1)
        adj = jnp.where(row == col, 0.0, adj)
        adj_s[...] = adj
        m1 = jnp.max(adj, axis=0, keepdims=True)
        masked = jnp.where(adj == m1, -3.0, adj)
        t = jnp.max(masked, axis=0, keepdims=True)
        tc_s[...] = t
        tr_s[...] = jnp.swapaxes(t, 0, 1)

    adj_j = adj_s[pl.ds(j * rows, rows), :]
    t_row = tr_s[pl.ds(j * rows, rows), :]
    t_col = tc_s[...]
    h = adj_j * 0.5
    o_ref[0] = (jnp.where(adj_j >= t_row, h, 0.0)
                + jnp.where(adj_j >= t_col, h, 0.0))


def kernel(x, W1, b1, W2, b2):
    b, m, d = x.shape
    rows = 512

    return pl.pallas_call(
        functools.partial(_fused_kernel, rows),
        grid=(b, m // rows),
        in_specs=[pl.BlockSpec((1, m, d), lambda i, j: (i, 0, 0))],
        out_specs=pl.BlockSpec((1, rows, m), lambda i, j: (i, j, 0)),
        out_shape=jax.ShapeDtypeStruct((b, m, m), jnp.float32),
        scratch_shapes=[
            pltpu.VMEM((m, m), jnp.float32),
            pltpu.VMEM((1, m), jnp.float32),
            pltpu.VMEM((m, 1), jnp.float32),
        ],
        compiler_params=pltpu.CompilerParams(
            dimension_semantics=("arbitrary", "arbitrary")),
    )(x)


# speedup vs baseline: 1.8355x; 1.2238x over previous
import functools

import jax
import jax.numpy as jnp
from jax.experimental import pallas as pl
from jax.experimental.pallas import tpu as pltpu


def _fused_kernel(rows, x_ref, o_ref, h_s, tc_s, tr_s):
    j = pl.program_id(1)
    m = x_ref.shape[1]

    @pl.when(j == 0)
    def _prologue():
        x = x_ref[0]
        norm = jnp.sqrt(jnp.sum(x * x, axis=-1, keepdims=True))
        xn = x / (jnp.maximum(norm, 1e-12) * jnp.sqrt(jnp.float32(2.0)))
        h = jax.lax.dot_general(xn, xn, (((1,), (1,)), ((), ())),
                                preferred_element_type=jnp.float32)
        h_s[...] = h
        eye_r = jax.lax.broadcasted_iota(jnp.int32, (128, 128), 0)
        eye_c = jax.lax.broadcasted_iota(jnp.int32, (128, 128), 1)
        eye = eye_r == eye_c
        for g in range(m // 128):
            s = slice(g * 128, (g + 1) * 128)
            h_s[s, s] = jnp.where(eye, 0.0, h_s[s, s])
        a = h_s[0:128, :]
        b = jnp.full_like(a, -3.0)
        for g in range(1, m // 128):
            v = h_s[g * 128:(g + 1) * 128, :]
            b = jnp.maximum(b, jnp.minimum(a, v))
            a = jnp.maximum(a, v)
        m1 = jnp.max(a, axis=0, keepdims=True)
        m2a = jnp.max(jnp.where(a == m1, -3.0, a), axis=0, keepdims=True)
        t = jnp.maximum(m2a, jnp.max(b, axis=0, keepdims=True))
        tc_s[...] = t
        tr_s[...] = jnp.swapaxes(t, 0, 1)

    h_j = h_s[pl.ds(j * rows, rows), :]
    t_row = tr_s[pl.ds(j * rows, rows), :]
    t_col = tc_s[...]
    o_ref[0] = (jnp.where(h_j >= t_row, h_j, 0.0)
                + jnp.where(h_j >= t_col, h_j, 0.0))


def kernel(x, W1, b1, W2, b2):
    b, m, d = x.shape
    rows = 512

    return pl.pallas_call(
        functools.partial(_fused_kernel, rows),
        grid=(b, m // rows),
        in_specs=[pl.BlockSpec((1, m, d), lambda i, j: (i, 0, 0))],
        out_specs=pl.BlockSpec((1, rows, m), lambda i, j: (i, j, 0)),
        out_shape=jax.ShapeDtypeStruct((b, m, m), jnp.float32),
        scratch_shapes=[
            pltpu.VMEM((m, m), jnp.float32),
            pltpu.VMEM((1, m), jnp.float32),
            pltpu.VMEM((m, 1), jnp.float32),
        ],
        compiler_params=pltpu.CompilerParams(
            dimension_semantics=("arbitrary", "arbitrary")),
    )(x)
